# baseline (speedup 1.0000x reference)
import jax
import jax.numpy as jnp
from jax import lax
from jax.experimental import pallas as pl
from jax.experimental.pallas import tpu as pltpu
from jax.experimental.pallas import tpu_sc as plsc

_L = 100
_B = 16


def _sc_body(alpha_ref, out_ref, a_v, f_v):
  wid = lax.axis_index("s") * 2 + lax.axis_index("c")

  @pl.when(wid < 1)
  def _():
    pltpu.sync_copy(alpha_ref.at[pl.ds(0, 1)], a_v)
    f_v[...] = a_v[0]
    pltpu.sync_copy(f_v, out_ref.at[0])


_sc_call = pl.kernel(
    _sc_body,
    out_type=jax.ShapeDtypeStruct((1, _B), jnp.float32),
    mesh=plsc.VectorSubcoreMesh(core_axis_name="c", subcore_axis_name="s",
                                num_cores=2, num_subcores=16),
    scratch_types=[
        pltpu.VMEM((1, _B), jnp.float32),
        pltpu.VMEM((_B,), jnp.float32),
    ],
)


def kernel(alpha):
  out = _sc_call(alpha)
  z = jnp.zeros((1, _L), jnp.float32)
  return ((z + out[0, 0]).astype(jnp.int32), z + out[0, 1], z + out[0, 2])


# speedup vs baseline: 1.0748x; 1.0748x over previous
import jax
import jax.numpy as jnp
from jax import lax
from jax.experimental import pallas as pl
from jax.experimental.pallas import tpu as pltpu
from jax.experimental.pallas import tpu_sc as plsc

_L = 100
_B = 16
_RPW = 4
_NW = 25

_NUM_CORES = 2
_NUM_SUBCORES = 16

_LOG16 = 2.772588722239781
_CHORD = _LOG16 / 15.0


def _allreduce(op, x, lane):
  for k in (1, 2, 4, 8):
    x = op(x, jnp.take(x, lane ^ k))
  return x


def _sc_body(alpha_ref, g_ref, arcs_ref, lp_ref, ent_ref,
             a_v, g_v, arcs_s, lp_s, ent_s, sem):
  wid = lax.axis_index("s") * _NUM_CORES + lax.axis_index("c")

  @pl.when(wid < _NW)
  def _():
    base = wid * _RPW
    in_a = pltpu.async_copy(alpha_ref.at[pl.ds(base, _RPW)], a_v, sem)
    in_g = pltpu.async_copy(g_ref.at[pl.ds(base, _RPW)], g_v, sem)
    in_a.wait()
    in_g.wait()

    lane = lax.iota(jnp.int32, _B)
    arcs_acc = jnp.zeros((_B,), jnp.int32)
    lp_acc = jnp.zeros((_B,), jnp.float32)
    ent_acc = jnp.zeros((_B,), jnp.float32)
    for j in range(_RPW):
      a = a_v[j]
      v = a + g_v[j]
      vmax = _allreduce(jnp.maximum, v, lane)
      idx = _allreduce(jnp.minimum, jnp.where(v == vmax, lane, _B), lane)
      am = a - _allreduce(jnp.maximum, a, lane)
      e = jnp.exp(am)
      s = _allreduce(jnp.add, e, lane)
      y = _CHORD * (s - 1.0)
      for _ in range(5):
        y = y + (s * jnp.exp(0.0 - y) - 1.0)
      logp = am - y
      lp = jnp.take(logp, idx)
      ent = 0.0 - _allreduce(jnp.add, (e / s) * logp, lane)
      at_j = lane == j
      arcs_acc = jnp.where(at_j, idx, arcs_acc)
      lp_acc = jnp.where(at_j, lp, lp_acc)
      ent_acc = jnp.where(at_j, ent, ent_acc)

    arcs_s[...] = arcs_acc
    lp_s[...] = lp_acc
    ent_s[...] = ent_acc
    out_a = pltpu.async_copy(arcs_s, arcs_ref.at[wid], sem)
    out_l = pltpu.async_copy(lp_s, lp_ref.at[wid], sem)
    out_e = pltpu.async_copy(ent_s, ent_ref.at[wid], sem)
    out_a.wait()
    out_l.wait()
    out_e.wait()


_sc_call = pl.kernel(
    _sc_body,
    out_type=(
        jax.ShapeDtypeStruct((_NW, _B), jnp.int32),
        jax.ShapeDtypeStruct((_NW, _B), jnp.float32),
        jax.ShapeDtypeStruct((_NW, _B), jnp.float32),
    ),
    mesh=plsc.VectorSubcoreMesh(core_axis_name="c", subcore_axis_name="s",
                                num_cores=_NUM_CORES,
                                num_subcores=_NUM_SUBCORES),
    scratch_types=[
        pltpu.VMEM((_RPW, _B), jnp.float32),
        pltpu.VMEM((_RPW, _B), jnp.float32),
        pltpu.VMEM((_B,), jnp.int32),
        pltpu.VMEM((_B,), jnp.float32),
        pltpu.VMEM((_B,), jnp.float32),
        pltpu.SemaphoreType.DMA,
    ],
)


def kernel(alpha):
  g = jax.random.gumbel(jax.random.key(42), (_L, _B), jnp.float32)
  arcs, lp, ent = _sc_call(alpha, g)
  arcs = arcs[:, :_RPW].reshape(_L)[None, :]
  lp = lp[:, :_RPW].reshape(_L)[None, :]
  ent = ent[:, :_RPW].reshape(_L)[None, :]
  return (arcs, lp, ent)
